# baseline (speedup 1.0000x reference)
import functools

import numpy as np

import jax
import jax.numpy as jnp
from jax import lax
from jax.experimental import pallas as pl
from jax.experimental.pallas import tpu as pltpu
from jax.experimental.pallas import tpu_sc as plsc

N_NODES = 10000
D = 128
DH = D // 2
E = 320000

NC = 2
NS = 16
NW = NC * NS
CH = 128
NCHUNK = 80
NCHUNK2 = 2 * NCHUNK
EPAD = NW * NCHUNK * CH
NPAD = 10240
RPT = NPAD // NS
ZROWS = 64

_HIGH = jax.lax.Precision.HIGHEST


def _mesh():
    return plsc.VectorSubcoreMesh(core_axis_name="c", subcore_axis_name="s")


def _sc_deg(dst3):
    @functools.partial(
        pl.kernel,
        mesh=_mesh(),
        compiler_params=pltpu.CompilerParams(use_tc_tiling_on_sc=False),
        out_type=jax.ShapeDtypeStruct((NC, NPAD, 16), jnp.float32),
        scratch_types=[
            pltpu.VMEM((NCHUNK, CH), jnp.int32),
            pltpu.VMEM((CH, 16), jnp.float32),
            pltpu.VMEM((ZROWS, 16), jnp.float32),
            pltpu.VMEM_SHARED((NPAD, 16), jnp.float32),
        ],
    )
    def k(dst_hbm, out_hbm, dstv, ones_v, zbuf, acc):
        c = lax.axis_index("c")
        s = lax.axis_index("s")
        wid = s * NC + c

        pltpu.sync_copy(dst_hbm.at[wid], dstv)

        @pl.loop(0, CH)
        def _(r):
            ones_v[r, pl.ds(0, 16)] = jnp.ones((16,), jnp.float32)

        @pl.loop(0, ZROWS)
        def _(r):
            zbuf[r, pl.ds(0, 16)] = jnp.zeros((16,), jnp.float32)

        @pl.loop(0, RPT // ZROWS)
        def _(b):
            pltpu.sync_copy(zbuf, acc.at[pl.ds(s * RPT + b * ZROWS, ZROWS)])

        plsc.subcore_barrier()

        @pl.loop(0, NCHUNK)
        def _(j):
            pltpu.sync_copy(ones_v, acc.at[dstv.at[j]], add=True)

        plsc.subcore_barrier()
        pltpu.sync_copy(acc.at[pl.ds(s * RPT, RPT)],
                        out_hbm.at[c, pl.ds(s * RPT, RPT)])

    return k(dst3)


NB = 4
HC = NCHUNK2 // 2

_PERM = np.empty(DH, np.int64)
for _g in range(DH // 32):
    for _i in range(16):
        _PERM[32 * _g + 2 * _i] = 32 * _g + _i
        _PERM[32 * _g + 2 * _i + 1] = 32 * _g + 16 + _i


def _sc_agg(y2b, src2, dst2):
    @functools.partial(
        pl.kernel,
        mesh=_mesh(),
        compiler_params=pltpu.CompilerParams(use_tc_tiling_on_sc=False,
                                             needs_layout_passes=False),
        out_type=jax.ShapeDtypeStruct((NC, NPAD, DH), jnp.float32),
        scratch_types=[
            pltpu.VMEM((HC, CH), jnp.int32),
            pltpu.VMEM((HC, CH), jnp.int32),
            [pltpu.VMEM((CH, DH), jnp.bfloat16)] * NB,
            [pltpu.VMEM((CH, DH), jnp.float32)] * NB,
            pltpu.VMEM((ZROWS, DH), jnp.float32),
            pltpu.VMEM_SHARED((NPAD, DH), jnp.float32),
            [pltpu.SemaphoreType.DMA] * NB,
            [pltpu.SemaphoreType.DMA] * NB,
        ],
    )
    def k(y_hbm, src_hbm, dst_hbm, out_hbm, srcv, dstv, brows, frows, zbuf,
          acc, gsems, ssems):
        c = lax.axis_index("c")
        s = lax.axis_index("s")
        ytab = y_hbm.at[c]

        @pl.loop(0, ZROWS)
        def _(r):
            @pl.loop(0, DH // 16)
            def _(cc):
                zbuf[r, pl.ds(cc * 16, 16)] = jnp.zeros((16,), jnp.float32)

        @pl.loop(0, RPT // ZROWS)
        def _(b):
            pltpu.sync_copy(zbuf, acc.at[pl.ds(s * RPT + b * ZROWS, ZROWS)])

        plsc.subcore_barrier()

        def convert(b):
            @plsc.parallel_loop(0, CH, unroll=4)
            def _(r):
                for g in range(DH // 32):
                    ab = brows[b][r, pl.ds(g * 32, 32)]
                    lo, hi = plsc.unpack(
                        ab, format=plsc.PackFormat.INTERLEAVED,
                        preferred_element_type=jnp.float32)
                    frows[b][r, pl.ds(g * 32, 16)] = lo
                    frows[b][r, pl.ds(g * 32 + 16, 16)] = hi

        for h in range(2):
            pltpu.sync_copy(src_hbm.at[s, pl.ds(h * HC, HC)], srcv)
            pltpu.sync_copy(dst_hbm.at[s, pl.ds(h * HC, HC)], dstv)
            for b in range(NB):
                pltpu.async_copy(ytab.at[srcv.at[b]], brows[b], gsems[b])

            @pl.loop(0, HC, step=NB)
            def _(j):
                for b in range(NB):
                    pltpu.make_async_copy(ytab.at[srcv.at[j + b]], brows[b],
                                          gsems[b]).wait()

                    @pl.when(j > 0)
                    def _():
                        pltpu.make_async_copy(
                            frows[b], acc.at[dstv.at[0]], ssems[b]).wait()

                    convert(b)
                    pltpu.async_copy(frows[b], acc.at[dstv.at[j + b]],
                                     ssems[b], add=True)

                    @pl.when(j + b + NB < HC)
                    def _():
                        pltpu.async_copy(ytab.at[srcv.at[j + b + NB]],
                                         brows[b], gsems[b])

            for b in range(NB):
                pltpu.make_async_copy(frows[b], acc.at[dstv.at[0]],
                                      ssems[b]).wait()

        plsc.subcore_barrier()
        pltpu.sync_copy(acc.at[pl.ds(s * RPT, RPT)],
                        out_hbm.at[c, pl.ds(s * RPT, RPT)])

    return k(y2b, src2, dst2)


_BLK = 2000
_GRID = N_NODES // _BLK

_row_spec = pl.BlockSpec((_BLK, D), lambda i: (i, 0))
_half_spec = pl.BlockSpec((2, _BLK, DH), lambda i: (0, i, 0))
_deg_spec = pl.BlockSpec((2, _BLK, 16), lambda i: (0, i, 0))
_w_spec = pl.BlockSpec((D, D), lambda i: (0, 0))
_v_spec = pl.BlockSpec((1, D), lambda i: (0, 0))


def _split_store(y_ref, yn):
    y_ref[0, ...] = yn[:, :DH].astype(jnp.bfloat16)
    y_ref[1, ...] = yn[:, DH:].astype(jnp.bfloat16)


def _pre_body(h_ref, w_ref, y_ref):
    _split_store(y_ref, jnp.dot(h_ref[...], w_ref[...],
                                preferred_element_type=jnp.float32,
                                precision=_HIGH))


def _tc_pre(h, WlT):
    return pl.pallas_call(
        _pre_body,
        grid=(_GRID,),
        in_specs=[_row_spec, _w_spec],
        out_specs=_half_spec,
        out_shape=jax.ShapeDtypeStruct((NC, NPAD, DH), jnp.bfloat16),
    )(h, WlT)


def _mix(p_ref, dg_ref, h_ref, WrT_ref, bl_ref, a_ref):
    p = jnp.concatenate([p_ref[0, ...], p_ref[1, ...]], axis=-1)
    deg = dg_ref[0, :, 0:1] + dg_ref[1, :, 0:1]
    invd = 1.0 / jnp.maximum(deg, 1.0)
    v = (p * invd + bl_ref[...]
         + jnp.dot(h_ref[...], WrT_ref[...],
                   preferred_element_type=jnp.float32, precision=_HIGH))
    return jnp.where(v > 0, v, a_ref[...] * v)


def _fused_body(p_ref, dg_ref, h_ref, WrT_ref, bl_ref, a_ref, WlTn_ref,
                hn_ref, yn_ref):
    hn = _mix(p_ref, dg_ref, h_ref, WrT_ref, bl_ref, a_ref)
    hn_ref[...] = hn
    _split_store(yn_ref, jnp.dot(hn, WlTn_ref[...],
                                 preferred_element_type=jnp.float32,
                                 precision=_HIGH))


def _tc_fused(p, dg, h, WrT, bl, a, WlTn):
    return pl.pallas_call(
        _fused_body,
        grid=(_GRID,),
        in_specs=[_half_spec, _deg_spec, _row_spec, _w_spec, _v_spec, _v_spec,
                  _w_spec],
        out_specs=(_row_spec, _half_spec),
        out_shape=(jax.ShapeDtypeStruct((N_NODES, D), jnp.float32),
                   jax.ShapeDtypeStruct((NC, NPAD, DH), jnp.bfloat16)),
    )(p, dg, h, WrT, bl, a, WlTn)


def _final_body(p_ref, dg_ref, h_ref, WrT_ref, bl_ref, a_ref, hn_ref):
    hn_ref[...] = _mix(p_ref, dg_ref, h_ref, WrT_ref, bl_ref, a_ref)


def _tc_final(p, dg, h, WrT, bl, a):
    return pl.pallas_call(
        _final_body,
        grid=(_GRID,),
        in_specs=[_half_spec, _deg_spec, _row_spec, _w_spec, _v_spec, _v_spec],
        out_specs=_row_spec,
        out_shape=jax.ShapeDtypeStruct((N_NODES, D), jnp.float32),
    )(p, dg, h, WrT, bl, a)


def kernel(x, edge_index, Wl0, bl0, Wr0, a0, Wl1, bl1, Wr1, a1,
           Wl2, bl2, Wr2, a2):
    src = edge_index[0].astype(jnp.int32)
    dst = edge_index[1].astype(jnp.int32)
    src3 = jnp.concatenate(
        [src, jnp.zeros((EPAD - E,), jnp.int32)]).reshape(NW, NCHUNK, CH)
    dst3 = jnp.concatenate(
        [dst, jnp.full((EPAD - E,), N_NODES, jnp.int32)]).reshape(NW, NCHUNK, CH)
    src2 = src3.reshape(NS, NCHUNK2, CH)
    dst2 = dst3.reshape(NS, NCHUNK2, CH)

    dg = _sc_deg(dst3)

    blv = [bl0.reshape(1, D), bl1.reshape(1, D), bl2.reshape(1, D)]
    av = [a0.reshape(1, D), a1.reshape(1, D), a2.reshape(1, D)]
    pfull = np.concatenate([_PERM, _PERM + DH])
    WlT = [Wl0.T[:, pfull], Wl1.T[:, pfull], Wl2.T[:, pfull]]
    WrT = [Wr0.T, Wr1.T, Wr2.T]

    y = _tc_pre(x, WlT[0])
    p = _sc_agg(y, src2, dst2)
    h, y = _tc_fused(p, dg, x, WrT[0], blv[0], av[0], WlT[1])
    p = _sc_agg(y, src2, dst2)
    h, y = _tc_fused(p, dg, h, WrT[1], blv[1], av[1], WlT[2])
    p = _sc_agg(y, src2, dst2)
    return _tc_final(p, dg, h, WrT[2], blv[2], av[2])


# speedup vs baseline: 1.0241x; 1.0241x over previous
import functools

import numpy as np

import jax
import jax.numpy as jnp
from jax import lax
from jax.experimental import pallas as pl
from jax.experimental.pallas import tpu as pltpu
from jax.experimental.pallas import tpu_sc as plsc

N_NODES = 10000
D = 128
DH = D // 2
E = 320000

NC = 2
NS = 16
NW = NC * NS
CH = 128
NCHUNK = 80
NCHUNK2 = 2 * NCHUNK
EPAD = NW * NCHUNK * CH
NPAD = 10240
RPT = NPAD // NS
ZROWS = 128

_HIGH = jax.lax.Precision.HIGHEST


def _mesh():
    return plsc.VectorSubcoreMesh(core_axis_name="c", subcore_axis_name="s")


def _sc_deg(dst3):
    @functools.partial(
        pl.kernel,
        mesh=_mesh(),
        compiler_params=pltpu.CompilerParams(use_tc_tiling_on_sc=False),
        out_type=jax.ShapeDtypeStruct((NC, NPAD, 16), jnp.float32),
        scratch_types=[
            pltpu.VMEM((NCHUNK, CH), jnp.int32),
            pltpu.VMEM((CH, 16), jnp.float32),
            pltpu.VMEM((ZROWS, 16), jnp.float32),
            pltpu.VMEM_SHARED((NPAD, 16), jnp.float32),
        ],
    )
    def k(dst_hbm, out_hbm, dstv, ones_v, zbuf, acc):
        c = lax.axis_index("c")
        s = lax.axis_index("s")
        wid = s * NC + c

        pltpu.sync_copy(dst_hbm.at[wid], dstv)

        @pl.loop(0, CH)
        def _(r):
            ones_v[r, pl.ds(0, 16)] = jnp.ones((16,), jnp.float32)

        @pl.loop(0, ZROWS)
        def _(r):
            zbuf[r, pl.ds(0, 16)] = jnp.zeros((16,), jnp.float32)

        @pl.loop(0, RPT // ZROWS)
        def _(b):
            pltpu.sync_copy(zbuf, acc.at[pl.ds(s * RPT + b * ZROWS, ZROWS)])

        plsc.subcore_barrier()

        @pl.loop(0, NCHUNK)
        def _(j):
            pltpu.sync_copy(ones_v, acc.at[dstv.at[j]], add=True)

        plsc.subcore_barrier()
        pltpu.sync_copy(acc.at[pl.ds(s * RPT, RPT)],
                        out_hbm.at[c, pl.ds(s * RPT, RPT)])

    return k(dst3)


NB = 4
HC = NCHUNK2 // 2

_PERM = np.empty(DH, np.int64)
for _g in range(DH // 32):
    for _i in range(16):
        _PERM[32 * _g + 2 * _i] = 32 * _g + _i
        _PERM[32 * _g + 2 * _i + 1] = 32 * _g + 16 + _i


def _sc_agg(y2b, src2, dst2):
    @functools.partial(
        pl.kernel,
        mesh=_mesh(),
        compiler_params=pltpu.CompilerParams(use_tc_tiling_on_sc=False,
                                             needs_layout_passes=False),
        out_type=jax.ShapeDtypeStruct((NC, NPAD, DH), jnp.float32),
        scratch_types=[
            pltpu.VMEM((HC, CH), jnp.int32),
            pltpu.VMEM((HC, CH), jnp.int32),
            [pltpu.VMEM((CH, DH), jnp.bfloat16)] * NB,
            [pltpu.VMEM((CH, DH), jnp.float32)] * NB,
            pltpu.VMEM((ZROWS, DH), jnp.float32),
            pltpu.VMEM_SHARED((NPAD, DH), jnp.float32),
            [pltpu.SemaphoreType.DMA] * NB,
            [pltpu.SemaphoreType.DMA] * NB,
        ],
    )
    def k(y_hbm, src_hbm, dst_hbm, out_hbm, srcv, dstv, brows, frows, zbuf,
          acc, gsems, ssems):
        c = lax.axis_index("c")
        s = lax.axis_index("s")
        ytab = y_hbm.at[c]

        @pl.loop(0, ZROWS)
        def _(r):
            @pl.loop(0, DH // 16)
            def _(cc):
                zbuf[r, pl.ds(cc * 16, 16)] = jnp.zeros((16,), jnp.float32)

        @pl.loop(0, RPT // ZROWS)
        def _(b):
            pltpu.sync_copy(zbuf, acc.at[pl.ds(s * RPT + b * ZROWS, ZROWS)])

        plsc.subcore_barrier()

        def convert(b):
            @plsc.parallel_loop(0, CH, unroll=4)
            def _(r):
                for g in range(DH // 32):
                    ab = brows[b][r, pl.ds(g * 32, 32)]
                    lo, hi = plsc.unpack(
                        ab, format=plsc.PackFormat.INTERLEAVED,
                        preferred_element_type=jnp.float32)
                    frows[b][r, pl.ds(g * 32, 16)] = lo
                    frows[b][r, pl.ds(g * 32 + 16, 16)] = hi

        for h in range(2):
            pltpu.sync_copy(src_hbm.at[s, pl.ds(h * HC, HC)], srcv)
            pltpu.sync_copy(dst_hbm.at[s, pl.ds(h * HC, HC)], dstv)
            for b in range(NB):
                pltpu.async_copy(ytab.at[srcv.at[b]], brows[b], gsems[b])

            @pl.loop(0, HC, step=NB)
            def _(j):
                for b in range(NB):
                    pltpu.make_async_copy(ytab.at[srcv.at[j + b]], brows[b],
                                          gsems[b]).wait()

                    @pl.when(j > 0)
                    def _():
                        pltpu.make_async_copy(
                            frows[b], acc.at[dstv.at[0]], ssems[b]).wait()

                    convert(b)
                    pltpu.async_copy(frows[b], acc.at[dstv.at[j + b]],
                                     ssems[b], add=True)

                    @pl.when(j + b + NB < HC)
                    def _():
                        pltpu.async_copy(ytab.at[srcv.at[j + b + NB]],
                                         brows[b], gsems[b])

            for b in range(NB):
                pltpu.make_async_copy(frows[b], acc.at[dstv.at[0]],
                                      ssems[b]).wait()

        plsc.subcore_barrier()
        pltpu.sync_copy(acc.at[pl.ds(s * RPT, RPT)],
                        out_hbm.at[c, pl.ds(s * RPT, RPT)])

    return k(y2b, src2, dst2)


_BLK = 1280
_GRID = NPAD // _BLK

_row_spec = pl.BlockSpec((_BLK, D), lambda i: (i, 0))
_half_spec = pl.BlockSpec((2, _BLK, DH), lambda i: (0, i, 0))
_deg_spec = pl.BlockSpec((2, _BLK, 16), lambda i: (0, i, 0))
_w_spec = pl.BlockSpec((D, D), lambda i: (0, 0))
_v_spec = pl.BlockSpec((1, D), lambda i: (0, 0))


def _split_store(y_ref, yn):
    y_ref[0, ...] = yn[:, :DH].astype(jnp.bfloat16)
    y_ref[1, ...] = yn[:, DH:].astype(jnp.bfloat16)


def _pre_body(h_ref, w_ref, y_ref):
    _split_store(y_ref, jnp.dot(h_ref[...], w_ref[...],
                                preferred_element_type=jnp.float32,
                                precision=_HIGH))


def _tc_pre(h, WlT):
    return pl.pallas_call(
        _pre_body,
        grid=(_GRID,),
        in_specs=[_row_spec, _w_spec],
        out_specs=_half_spec,
        out_shape=jax.ShapeDtypeStruct((NC, NPAD, DH), jnp.bfloat16),
    )(h, WlT)


def _mix(p_ref, dg_ref, h_ref, WrT_ref, bl_ref, a_ref):
    p = jnp.concatenate([p_ref[0, ...], p_ref[1, ...]], axis=-1)
    deg = dg_ref[0, :, 0:1] + dg_ref[1, :, 0:1]
    invd = 1.0 / jnp.maximum(deg, 1.0)
    v = (p * invd + bl_ref[...]
         + jnp.dot(h_ref[...], WrT_ref[...],
                   preferred_element_type=jnp.float32, precision=_HIGH))
    return jnp.where(v > 0, v, a_ref[...] * v)


def _fused_body(p_ref, dg_ref, h_ref, WrT_ref, bl_ref, a_ref, WlTn_ref,
                hn_ref, yn_ref):
    hn = _mix(p_ref, dg_ref, h_ref, WrT_ref, bl_ref, a_ref)
    hn_ref[...] = hn
    _split_store(yn_ref, jnp.dot(hn, WlTn_ref[...],
                                 preferred_element_type=jnp.float32,
                                 precision=_HIGH))


def _tc_fused(p, dg, h, WrT, bl, a, WlTn):
    return pl.pallas_call(
        _fused_body,
        grid=(_GRID,),
        in_specs=[_half_spec, _deg_spec, _row_spec, _w_spec, _v_spec, _v_spec,
                  _w_spec],
        out_specs=(_row_spec, _half_spec),
        out_shape=(jax.ShapeDtypeStruct((NPAD, D), jnp.float32),
                   jax.ShapeDtypeStruct((NC, NPAD, DH), jnp.bfloat16)),
    )(p, dg, h, WrT, bl, a, WlTn)


def _final_body(p_ref, dg_ref, h_ref, WrT_ref, bl_ref, a_ref, hn_ref):
    hn_ref[...] = _mix(p_ref, dg_ref, h_ref, WrT_ref, bl_ref, a_ref)


def _tc_final(p, dg, h, WrT, bl, a):
    return pl.pallas_call(
        _final_body,
        grid=(_GRID,),
        in_specs=[_half_spec, _deg_spec, _row_spec, _w_spec, _v_spec, _v_spec],
        out_specs=_row_spec,
        out_shape=jax.ShapeDtypeStruct((NPAD, D), jnp.float32),
    )(p, dg, h, WrT, bl, a)


def kernel(x, edge_index, Wl0, bl0, Wr0, a0, Wl1, bl1, Wr1, a1,
           Wl2, bl2, Wr2, a2):
    src = edge_index[0].astype(jnp.int32)
    dst = edge_index[1].astype(jnp.int32)
    src3 = jnp.concatenate(
        [src, jnp.zeros((EPAD - E,), jnp.int32)]).reshape(NW, NCHUNK, CH)
    dst3 = jnp.concatenate(
        [dst, jnp.full((EPAD - E,), N_NODES, jnp.int32)]).reshape(NW, NCHUNK, CH)
    src2 = src3.reshape(NS, NCHUNK2, CH)
    dst2 = dst3.reshape(NS, NCHUNK2, CH)

    x_p = jnp.pad(x, ((0, NPAD - N_NODES), (0, 0)))

    dg = _sc_deg(dst3)

    blv = [bl0.reshape(1, D), bl1.reshape(1, D), bl2.reshape(1, D)]
    av = [a0.reshape(1, D), a1.reshape(1, D), a2.reshape(1, D)]
    pfull = np.concatenate([_PERM, _PERM + DH])
    WlT = [Wl0.T[:, pfull], Wl1.T[:, pfull], Wl2.T[:, pfull]]
    WrT = [Wr0.T, Wr1.T, Wr2.T]

    y = _tc_pre(x_p, WlT[0])
    p = _sc_agg(y, src2, dst2)
    h, y = _tc_fused(p, dg, x_p, WrT[0], blv[0], av[0], WlT[1])
    p = _sc_agg(y, src2, dst2)
    h, y = _tc_fused(p, dg, h, WrT[1], blv[1], av[1], WlT[2])
    p = _sc_agg(y, src2, dst2)
    h = _tc_final(p, dg, h, WrT[2], blv[2], av[2])

    return h[:N_NODES]
